# baseline (speedup 1.0000x reference)
import functools

import jax
import jax.numpy as jnp
from jax import lax
from jax.experimental import pallas as pl
from jax.experimental.pallas import tpu as pltpu
from jax.experimental.pallas import tpu_sc as plsc

N, K, E, O, I = 2048, 2, 64, 128, 128
M = O * I

TN = 256

NC, NS = 2, 16
NW = NC * NS
TW = N // NW
CH = 16
NCH = TW // CH
OP = O + 1
LANES = 16


def _tc_weight_kernel(wp_ref, wi_ref, bank_ref, out_ref):
    wp = wp_ref[...]
    wi = wi_ref[...]
    iota = lax.broadcasted_iota(jnp.int32, (TN, E), 1)
    s = (wp[:, 0:1] * (wi[:, 0:1] == iota).astype(jnp.float32)
         + wp[:, 1:2] * (wi[:, 1:2] == iota).astype(jnp.float32))
    bank = bank_ref[...].reshape(E, M)
    res = jnp.dot(s, bank, preferred_element_type=jnp.float32)
    out_ref[...] = res.reshape(TN, O, I)


def _sc_bias_body(p0_hbm, p1_hbm, i0_hbm, i1_hbm, bankp_hbm, out_hbm,
                  bank_v, i0_v, i1_v, p0_v, p1_v, outb0, outb1,
                  sem_b0, sem_b1, sem_in):
    wid = lax.axis_index("s") * NC + lax.axis_index("c")
    t0 = wid * TW

    pltpu.async_copy(bankp_hbm, bank_v, sem_in).wait()
    pltpu.async_copy(i0_hbm.at[pl.ds(t0, TW)], i0_v, sem_in).wait()
    pltpu.async_copy(i1_hbm.at[pl.ds(t0, TW)], i1_v, sem_in).wait()
    pltpu.async_copy(p0_hbm.at[pl.ds(t0, TW)], p0_v, sem_in).wait()
    pltpu.async_copy(p1_hbm.at[pl.ds(t0, TW)], p1_v, sem_in).wait()

    lanes = lax.iota(jnp.int32, LANES)
    zeros = jnp.zeros((LANES,), jnp.int32)

    def chunk(ch, outb, sem, first):
        c0 = ch * CH

        @pl.when(jnp.logical_not(first))
        def _():
            pltpu.make_async_copy(
                outb.at[:, pl.ds(0, O)],
                out_hbm.at[pl.ds(t0 + c0, CH), pl.ds(0, O)], sem).wait()

        i0v = i0_v[pl.ds(c0, CH)]
        i1v = i1_v[pl.ds(c0, CH)]
        p0v = p0_v[pl.ds(c0, CH)]
        p1v = p1_v[pl.ds(c0, CH)]
        a0 = i0v * OP
        b0 = i1v * OP

        @plsc.parallel_loop(0, O, 1, unroll=8, carry=(a0, b0, zeros))
        def _jloop(j, c):
            aa, bb, oo = c
            va = plsc.load_gather(bank_v, [aa])
            vb = plsc.load_gather(bank_v, [bb])
            plsc.store_scatter(outb, [lanes, oo], p0v * va + p1v * vb)
            return (aa + 1, bb + 1, oo + 1)

        pltpu.async_copy(
            outb.at[:, pl.ds(0, O)],
            out_hbm.at[pl.ds(t0 + c0, CH), pl.ds(0, O)], sem)

    def pair(g, carry):
        chunk(g * 2, outb0, sem_b0, g == 0)
        chunk(g * 2 + 1, outb1, sem_b1, g == 0)
        return carry

    lax.fori_loop(0, NCH // 2, pair, 0, unroll=1)
    pltpu.make_async_copy(
        outb0.at[:, pl.ds(0, O)],
        out_hbm.at[pl.ds(t0, CH), pl.ds(0, O)], sem_b0).wait()
    pltpu.make_async_copy(
        outb1.at[:, pl.ds(0, O)],
        out_hbm.at[pl.ds(t0, CH), pl.ds(0, O)], sem_b1).wait()


_sc_bias = functools.partial(
    pl.kernel,
    out_type=jax.ShapeDtypeStruct((N, O), jnp.float32),
    mesh=plsc.VectorSubcoreMesh(core_axis_name="c", subcore_axis_name="s"),
    compiler_params=pltpu.CompilerParams(needs_layout_passes=False),
    scratch_types=[
        pltpu.VMEM((E * OP,), jnp.float32),
        pltpu.VMEM((TW,), jnp.int32),
        pltpu.VMEM((TW,), jnp.int32),
        pltpu.VMEM((TW,), jnp.float32),
        pltpu.VMEM((TW,), jnp.float32),
        pltpu.VMEM((CH, OP), jnp.float32),
        pltpu.VMEM((CH, OP), jnp.float32),
        pltpu.SemaphoreType.DMA,
        pltpu.SemaphoreType.DMA,
        pltpu.SemaphoreType.DMA,
    ],
)(_sc_bias_body)


def kernel(weight_probs, weight_indices, bias_probs, bias_indices,
           weight_bank, bias_bank):
    wi = weight_indices.astype(jnp.int32)
    bi = bias_indices.astype(jnp.int32)

    bankp = jnp.pad(bias_bank, ((0, 0), (0, OP - O))).reshape(E * OP)

    bout = _sc_bias(bias_probs[:, 0], bias_probs[:, 1],
                    bi[:, 0], bi[:, 1], bankp)

    out = pl.pallas_call(
        _tc_weight_kernel,
        grid=(N // TN,),
        in_specs=[
            pl.BlockSpec((TN, K), lambda i: (i, 0)),
            pl.BlockSpec((TN, K), lambda i: (i, 0)),
            pl.BlockSpec((E, O, I), lambda i: (0, 0, 0)),
        ],
        out_specs=pl.BlockSpec((TN, O, I), lambda i: (i, 0, 0)),
        out_shape=jax.ShapeDtypeStruct((N, O, I), jnp.float32),
    )(weight_probs, wi, weight_bank)

    return out, bout


# speedup vs baseline: 1.2502x; 1.2502x over previous
import jax
import jax.numpy as jnp
from jax.experimental import pallas as pl

N, K, E, O, I = 2048, 2, 64, 128, 128
M = O * I

TN = 256


def _mix_kernel(wp_ref, wi_ref, bp_ref, bi_ref, bank_ref, bbank_ref,
                out_ref, bout_ref):
    wp = wp_ref[...]
    wi = wi_ref[...]
    iota = jax.lax.broadcasted_iota(jnp.int32, (TN, E), 1)
    s = (wp[:, 0:1] * (wi[:, 0:1] == iota).astype(jnp.float32)
         + wp[:, 1:2] * (wi[:, 1:2] == iota).astype(jnp.float32))
    bank = bank_ref[...].reshape(E, M)
    res = jnp.dot(s, bank, preferred_element_type=jnp.float32)
    out_ref[...] = res.reshape(TN, O, I)

    bp = bp_ref[...]
    bi = bi_ref[...]
    sb = (bp[:, 0:1] * (bi[:, 0:1] == iota).astype(jnp.float32)
          + bp[:, 1:2] * (bi[:, 1:2] == iota).astype(jnp.float32))
    bout_ref[...] = jnp.dot(sb, bbank_ref[...],
                            preferred_element_type=jnp.float32)


def kernel(weight_probs, weight_indices, bias_probs, bias_indices,
           weight_bank, bias_bank):
    wi = weight_indices.astype(jnp.int32)
    bi = bias_indices.astype(jnp.int32)

    grid = (N // TN,)
    out, bout = pl.pallas_call(
        _mix_kernel,
        grid=grid,
        in_specs=[
            pl.BlockSpec((TN, K), lambda i: (i, 0)),
            pl.BlockSpec((TN, K), lambda i: (i, 0)),
            pl.BlockSpec((TN, K), lambda i: (i, 0)),
            pl.BlockSpec((TN, K), lambda i: (i, 0)),
            pl.BlockSpec((E, O, I), lambda i: (0, 0, 0)),
            pl.BlockSpec((E, O), lambda i: (0, 0)),
        ],
        out_specs=[
            pl.BlockSpec((TN, O, I), lambda i: (i, 0, 0)),
            pl.BlockSpec((TN, O), lambda i: (i, 0)),
        ],
        out_shape=[
            jax.ShapeDtypeStruct((N, O, I), jnp.float32),
            jax.ShapeDtypeStruct((N, O), jnp.float32),
        ],
    )(weight_probs, wi, bias_probs, bi, weight_bank, bias_bank)

    return out, bout
